# baseline (speedup 1.0000x reference)
import functools

import jax
import jax.numpy as jnp
from jax import lax
from jax.experimental import pallas as pl
from jax.experimental.pallas import tpu as pltpu
from jax.experimental.pallas import tpu_sc as plsc

_P = 0.1
_SEED = 42
_LANES = 16
_CHUNK = 128


@functools.lru_cache(maxsize=None)
def _compaction_kernel(B: int, L: int, V: int):
    mesh = plsc.VectorSubcoreMesh(core_axis_name="c", subcore_axis_name="s")
    n_vecs = L // _LANES

    @functools.partial(
        pl.kernel,
        mesh=mesh,
        out_type=jax.ShapeDtypeStruct((B * L, V), jnp.float32),
        scratch_types=[
            pltpu.VMEM((L,), jnp.int32),
            pltpu.VMEM((L,), jnp.int32),
            pltpu.VMEM((_CHUNK, V), jnp.float32),
            pltpu.VMEM((_CHUNK, V), jnp.float32),
            pltpu.SemaphoreType.DMA,
        ],
        compiler_params=pltpu.CompilerParams(
            needs_layout_passes=False, use_tc_tiling_on_sc=False),
    )
    def kern(msg_hbm, keep_hbm, eos_hbm, out_hbm, keep_v, src_v, buf_v, eos_v, sem):
        cid = lax.axis_index("c")
        sid = lax.axis_index("s")
        wid = sid * 2 + cid

        @pl.when(wid < B)
        def _():
            b = wid
            base = b * L
            pltpu.sync_copy(keep_hbm.at[b], keep_v)
            pltpu.sync_copy(eos_hbm, eos_v)

            def fill(i, c):
                src_v[pl.ds(i * _LANES, _LANES)] = jnp.full(
                    (_LANES,), base, jnp.int32)
                return c

            lax.fori_loop(0, n_vecs, fill, 0)

            def scan_step(i, cnt):
                kv = keep_v[pl.ds(i * _LANES, _LANES)]
                s = jnp.cumsum(kv)
                slots = s + (cnt - 1)
                pos = base + i * _LANES + lax.iota(jnp.int32, _LANES)
                plsc.store_scatter(src_v, [slots], pos, mask=kv > 0)
                return cnt + jnp.max(s)

            num_kept = lax.fori_loop(0, n_vecs, scan_step, jnp.int32(0))

            n_full = num_kept // _CHUNK

            def gather_step(i, c):
                idx = src_v.at[pl.ds(i * _CHUNK, _CHUNK)]
                pltpu.async_copy(msg_hbm.at[idx], buf_v, sem).wait()
                pltpu.sync_copy(
                    buf_v, out_hbm.at[pl.ds(base + i * _CHUNK, _CHUNK)])
                return c

            lax.fori_loop(0, n_full, gather_step, 0)

            c0 = num_kept - n_full * _CHUNK
            eos_head = jnp.where(
                lax.iota(jnp.int32, _LANES) == 0, 1.0, 0.0
            ).astype(jnp.float32)
            eos_zero = jnp.zeros((_LANES,), jnp.float32)

            @pl.when(c0 > 0)
            def _boundary():
                idx = src_v.at[pl.ds(n_full * _CHUNK, _CHUNK)]
                pltpu.async_copy(msg_hbm.at[idx], buf_v, sem).wait()

                def fix(j, c):
                    buf_v[j, pl.ds(0, _LANES)] = eos_head
                    for k in range(1, V // _LANES):
                        buf_v[j, pl.ds(k * _LANES, _LANES)] = eos_zero
                    return c

                lax.fori_loop(c0, _CHUNK, fix, 0)
                pltpu.sync_copy(
                    buf_v, out_hbm.at[pl.ds(base + n_full * _CHUNK, _CHUNK)])

            pad0 = n_full + jnp.where(c0 > 0, 1, 0)

            def pad_step(i, c):
                pltpu.sync_copy(eos_v, out_hbm.at[pl.ds(base + i * _CHUNK, _CHUNK)])
                return c

            lax.fori_loop(pad0, L // _CHUNK, pad_step, 0)

    return kern


def kernel(message, message_length, apply_noise):
    B, L, V = message.shape

    def noised():
        target = jax.random.uniform(jax.random.key(_SEED), (B, L)) < _P
        not_eosed = jnp.arange(L)[None, :] < (message_length - 1)[:, None]
        keep = 1 - jnp.logical_and(target, not_eosed).astype(jnp.int32)
        eos = jnp.zeros((_CHUNK, V), jnp.float32).at[:, 0].set(1.0)
        msg_flat = message.reshape(B * L, V)
        out = _compaction_kernel(B, L, V)(msg_flat, keep, eos)
        return out.reshape(B, L, V)

    return lax.cond(jnp.asarray(apply_noise) != 0, noised, lambda: message)
